# baseline (speedup 1.0000x reference)
import functools
import numpy as np
import jax
import jax.numpy as jnp
from jax import lax
from jax.experimental import pallas as pl
from jax.experimental.pallas import tpu as pltpu
from jax.experimental.pallas import tpu_sc as plsc

B = 8
N = 15
N_H = 5
C = 1024
H = 25
W = 25
NUM_CLASSES = 600
NUM_OBJ = 80
POOL = 7
SCALE = 1.0 / 32.0


def _pair_idx():
    xs, ys = np.meshgrid(np.arange(N), np.arange(N), indexing="ij")
    m = (xs != ys) & (xs < N_H)
    return xs[m].astype(np.int32), ys[m].astype(np.int32)


_XK, _YK = _pair_idx()
P = int(_XK.shape[0])

NC = 2
NS = 16
NW = NC * NS
ROWS = 2 * B * P
RPW = 40
NWA = ROWS // RPW
LANES = 16
CPAD = 640


def _roi_mt(crd):
    lt = jnp.minimum(crd[0:2], crd[4:6])
    rb = jnp.maximum(crd[2:4], crd[6:8])
    gx1 = lt[0:1] * SCALE - 0.5
    gy1 = lt[1:2] * SCALE - 0.5
    gx2 = rb[0:1] * SCALE - 0.5
    gy2 = rb[1:2] * SCALE - 0.5
    offi = lax.broadcasted_iota(jnp.int32, (POOL, 1), 0)
    off = (offi.astype(jnp.float32) + 0.5) / POOL
    px = gx1 + off * (gx2 - gx1)
    py = gy1 + off * (gy2 - gy1)

    def axis_weights(pv, size):
        f0 = jnp.floor(pv)
        frac = pv - f0
        i0 = jnp.clip(f0.astype(jnp.int32), 0, size - 1)
        i1 = jnp.clip(i0 + 1, 0, size - 1)
        pos = lax.broadcasted_iota(jnp.int32, (POOL, size, P), 1)
        w = (jnp.where(pos == i0[:, None, :], (1.0 - frac)[:, None, :], 0.0)
             + jnp.where(pos == i1[:, None, :], frac[:, None, :], 0.0))
        return w.sum(axis=0) * (1.0 / POOL)

    axT = axis_weights(px, W)
    ayT = axis_weights(py, H)
    return jnp.concatenate([ayT[y:y + 1, :] * axT for y in range(H)], axis=0)


def _tc_all_body(coords_ref, feat_ref, w1_ref, b1_ref, w2_ref, b2_ref,
                 wvp_ref, bvp_ref, scal_ref, logits_ref):
    ufs = []
    for b in range(B):
        mT = _roi_mt(coords_ref[b]).astype(jnp.bfloat16)
        ufs.append(lax.dot_general(feat_ref[b], mT, (((1,), (0,)), ((), ())),
                                   preferred_element_type=jnp.float32))
    ufT = jnp.concatenate(ufs, axis=1)
    ufb = ufT.astype(jnp.bfloat16)
    hT = jnp.maximum(
        lax.dot_general(w1_ref[...], ufb, (((1,), (0,)), ((), ())),
                        preferred_element_type=jnp.float32) + b1_ref[...], 0.0)
    mlpT = lax.dot_general(w2_ref[...], hT.astype(jnp.bfloat16),
                           (((1,), (0,)), ((), ())),
                           preferred_element_type=jnp.float32) + b2_ref[...]
    alpha = scal_ref[0, 0]
    mixT = alpha * mlpT + (1.0 - alpha) * ufT
    inv = 1.0 / jnp.sqrt(jnp.sum(mixT * mixT, axis=0, keepdims=True))
    normT = mixT * inv
    lg = lax.dot_general(normT.astype(jnp.bfloat16), wvp_ref[...],
                         (((0,), (1,)), ((), ())),
                         preferred_element_type=jnp.float32)
    logits_ref[...] = jnp.exp(scal_ref[0, 1]) * (lg + bvp_ref[...])


_SC_MESH = plsc.VectorSubcoreMesh(core_axis_name="c", subcore_axis_name="s")


@functools.partial(
    pl.kernel,
    out_type=jax.ShapeDtypeStruct((ROWS, NUM_CLASSES), jnp.float32),
    mesh=_SC_MESH,
    scratch_types=[
        pltpu.VMEM((RPW,), jnp.int32),
        pltpu.VMEM((RPW + LANES,), jnp.float32),
        pltpu.VMEM((RPW, CPAD), jnp.float32),
        pltpu.VMEM((RPW, NUM_CLASSES), jnp.float32),
        pltpu.SemaphoreType.DMA,
    ],
)
def _sc_prior(labels_hbm, s_hbm, table_hbm, out_hbm,
              idx_v, s_v, rows_v, dst_v, sem):
    wid = lax.axis_index("s") * NC + lax.axis_index("c")

    @pl.when(wid < NWA)
    def _():
        base = wid * RPW
        pltpu.sync_copy(labels_hbm.at[pl.ds(base, RPW)], idx_v)
        pltpu.sync_copy(s_hbm.at[pl.ds(base, RPW)], s_v.at[pl.ds(0, RPW)])
        s_v[pl.ds(RPW, LANES)] = jnp.zeros((LANES,), jnp.float32)
        pltpu.async_copy(table_hbm.at[idx_v], rows_v, sem).wait()

        offs = list(range(0, NUM_CLASSES - LANES + 1, LANES))
        if NUM_CLASSES % LANES:
            offs.append(NUM_CLASSES - LANES)

        for r in range(RPW):
            win = s_v[pl.ds(r, LANES)]
            sv = jnp.full((LANES,), win[0])
            for o in offs:
                dst_v[r, pl.ds(o, LANES)] = sv * rows_v[r, pl.ds(o, LANES)]
        pltpu.sync_copy(dst_v, out_hbm.at[pl.ds(base, RPW)])


def kernel(boxes, scores, labels, features, obj2target, W1, b1, W2, b2,
           Wvp, bvp, alpha1, logit_scale):
    xk = jnp.asarray(_XK)
    yk = jnp.asarray(_YK)
    sub = boxes[:, xk, :].transpose(0, 2, 1)
    obj = boxes[:, yk, :].transpose(0, 2, 1)
    coords = jnp.concatenate([sub, obj], axis=1)
    featr = features.reshape(B, C, H * W).astype(jnp.bfloat16)
    scal = jnp.stack([alpha1, logit_scale]).reshape(1, 2).astype(jnp.float32)

    lab_flat = labels[:, yk].astype(jnp.int32).reshape(-1)
    lab2 = jnp.tile(lab_flat, 2)
    sp = jnp.power(scores, 2.8)
    s_flat = jnp.concatenate([sp[:, xk].reshape(-1), sp[:, yk].reshape(-1)])
    table_pad = jnp.pad(obj2target, ((0, 0), (0, CPAD - NUM_CLASSES)))
    prior = _sc_prior(lab2, s_flat, table_pad)

    logits = pl.pallas_call(
        _tc_all_body,
        in_specs=[
            pl.BlockSpec((B, 8, P), lambda: (0, 0, 0)),
            pl.BlockSpec((B, C, H * W), lambda: (0, 0, 0)),
            pl.BlockSpec((C // 2, C), lambda: (0, 0)),
            pl.BlockSpec((C // 2, 1), lambda: (0, 0)),
            pl.BlockSpec((C, C // 2), lambda: (0, 0)),
            pl.BlockSpec((C, 1), lambda: (0, 0)),
            pl.BlockSpec((NUM_CLASSES, C), lambda: (0, 0)),
            pl.BlockSpec((1, NUM_CLASSES), lambda: (0, 0)),
            pl.BlockSpec((1, 2), lambda: (0, 0)),
        ],
        out_specs=pl.BlockSpec((B * P, NUM_CLASSES), lambda: (0, 0)),
        out_shape=jax.ShapeDtypeStruct((B * P, NUM_CLASSES), jnp.float32),
    )(coords, featr, W1.astype(jnp.bfloat16), b1.reshape(C // 2, 1),
      W2.astype(jnp.bfloat16), b2.reshape(C, 1),
      Wvp.astype(jnp.bfloat16), bvp.reshape(1, NUM_CLASSES), scal)

    return logits, prior.reshape(2, B, P, NUM_CLASSES)


# speedup vs baseline: 1.0517x; 1.0517x over previous
import functools
import numpy as np
import jax
import jax.numpy as jnp
from jax import lax
from jax.experimental import pallas as pl
from jax.experimental.pallas import tpu as pltpu
from jax.experimental.pallas import tpu_sc as plsc

B = 8
N = 15
N_H = 5
C = 1024
H = 25
W = 25
NUM_CLASSES = 600
NUM_OBJ = 80
POOL = 7
SCALE = 1.0 / 32.0


def _pair_idx():
    xs, ys = np.meshgrid(np.arange(N), np.arange(N), indexing="ij")
    m = (xs != ys) & (xs < N_H)
    return xs[m].astype(np.int32), ys[m].astype(np.int32)


_XK, _YK = _pair_idx()
P = int(_XK.shape[0])

NC = 2
NS = 16
NW = NC * NS
ROWS = 2 * B * P
RPW = 40
NWA = ROWS // RPW
LANES = 16
CPAD = 640


def _roi_mt(crd):
    lt = jnp.minimum(crd[0:2], crd[4:6])
    rb = jnp.maximum(crd[2:4], crd[6:8])
    gx1 = lt[0:1] * SCALE - 0.5
    gy1 = lt[1:2] * SCALE - 0.5
    gx2 = rb[0:1] * SCALE - 0.5
    gy2 = rb[1:2] * SCALE - 0.5
    offi = lax.broadcasted_iota(jnp.int32, (POOL, 1), 0)
    off = (offi.astype(jnp.float32) + 0.5) / POOL
    px = gx1 + off * (gx2 - gx1)
    py = gy1 + off * (gy2 - gy1)

    def axis_weights(pv, size):
        f0 = jnp.floor(pv)
        frac = pv - f0
        i0 = jnp.clip(f0.astype(jnp.int32), 0, size - 1)
        i1 = jnp.clip(i0 + 1, 0, size - 1)
        pos = lax.broadcasted_iota(jnp.int32, (POOL, size, P), 1)
        w = (jnp.where(pos == i0[:, None, :], (1.0 - frac)[:, None, :], 0.0)
             + jnp.where(pos == i1[:, None, :], frac[:, None, :], 0.0))
        return w.sum(axis=0) * (1.0 / POOL)

    axT = axis_weights(px, W)
    ayT = axis_weights(py, H)
    return jnp.concatenate([ayT[y:y + 1, :] * axT for y in range(H)], axis=0)


def _tc_all_body(coords_ref, feat_ref, w1_ref, b1_ref, w2_ref, b2_ref,
                 wvp_ref, bvp_ref, scal_ref, logits_ref):
    ufs = []
    for b in range(B):
        mT = _roi_mt(coords_ref[b])
        ufs.append(lax.dot_general(feat_ref[b], mT, (((1,), (0,)), ((), ())),
                                   preferred_element_type=jnp.float32))
    ufT = jnp.concatenate(ufs, axis=1)
    ufb = ufT.astype(jnp.bfloat16)
    hT = jnp.maximum(
        lax.dot_general(w1_ref[...], ufb, (((1,), (0,)), ((), ())),
                        preferred_element_type=jnp.float32) + b1_ref[...], 0.0)
    mlpT = lax.dot_general(w2_ref[...], hT.astype(jnp.bfloat16),
                           (((1,), (0,)), ((), ())),
                           preferred_element_type=jnp.float32) + b2_ref[...]
    alpha = scal_ref[0, 0]
    mixT = alpha * mlpT + (1.0 - alpha) * ufT
    inv = 1.0 / jnp.sqrt(jnp.sum(mixT * mixT, axis=0, keepdims=True))
    normT = mixT * inv
    lg = lax.dot_general(normT.astype(jnp.bfloat16), wvp_ref[...],
                         (((0,), (1,)), ((), ())),
                         preferred_element_type=jnp.float32)
    logits_ref[...] = jnp.exp(scal_ref[0, 1]) * (lg + bvp_ref[...])


_SC_MESH = plsc.VectorSubcoreMesh(core_axis_name="c", subcore_axis_name="s")


@functools.partial(
    pl.kernel,
    out_type=jax.ShapeDtypeStruct((ROWS, NUM_CLASSES), jnp.float32),
    mesh=_SC_MESH,
    scratch_types=[
        pltpu.VMEM((RPW,), jnp.int32),
        pltpu.VMEM((RPW + LANES,), jnp.float32),
        pltpu.VMEM((RPW, CPAD), jnp.float32),
        pltpu.VMEM((RPW, NUM_CLASSES), jnp.float32),
        pltpu.SemaphoreType.DMA,
    ],
)
def _sc_prior(labels_hbm, s_hbm, table_hbm, out_hbm,
              idx_v, s_v, rows_v, dst_v, sem):
    wid = lax.axis_index("s") * NC + lax.axis_index("c")

    @pl.when(wid < NWA)
    def _():
        base = wid * RPW
        pltpu.sync_copy(labels_hbm.at[pl.ds(base, RPW)], idx_v)
        pltpu.sync_copy(s_hbm.at[pl.ds(base, RPW)], s_v.at[pl.ds(0, RPW)])
        s_v[pl.ds(RPW, LANES)] = jnp.zeros((LANES,), jnp.float32)
        pltpu.async_copy(table_hbm.at[idx_v], rows_v, sem).wait()

        offs = list(range(0, NUM_CLASSES - LANES + 1, LANES))
        if NUM_CLASSES % LANES:
            offs.append(NUM_CLASSES - LANES)

        for r in range(RPW):
            win = s_v[pl.ds(r, LANES)]
            sv = jnp.full((LANES,), win[0])
            for o in offs:
                dst_v[r, pl.ds(o, LANES)] = sv * rows_v[r, pl.ds(o, LANES)]
        pltpu.sync_copy(dst_v, out_hbm.at[pl.ds(base, RPW)])


def kernel(boxes, scores, labels, features, obj2target, W1, b1, W2, b2,
           Wvp, bvp, alpha1, logit_scale):
    xk = jnp.asarray(_XK)
    yk = jnp.asarray(_YK)
    sub = boxes[:, xk, :].transpose(0, 2, 1)
    obj = boxes[:, yk, :].transpose(0, 2, 1)
    coords = jnp.concatenate([sub, obj], axis=1)
    featr = features.reshape(B, C, H * W)
    scal = jnp.stack([alpha1, logit_scale]).reshape(1, 2).astype(jnp.float32)

    lab_flat = labels[:, yk].astype(jnp.int32).reshape(-1)
    lab2 = jnp.tile(lab_flat, 2)
    sp = jnp.power(scores, 2.8)
    s_flat = jnp.concatenate([sp[:, xk].reshape(-1), sp[:, yk].reshape(-1)])
    table_pad = jnp.pad(obj2target, ((0, 0), (0, CPAD - NUM_CLASSES)))
    prior = _sc_prior(lab2, s_flat, table_pad)

    logits = pl.pallas_call(
        _tc_all_body,
        in_specs=[
            pl.BlockSpec((B, 8, P), lambda: (0, 0, 0)),
            pl.BlockSpec((B, C, H * W), lambda: (0, 0, 0)),
            pl.BlockSpec((C // 2, C), lambda: (0, 0)),
            pl.BlockSpec((C // 2, 1), lambda: (0, 0)),
            pl.BlockSpec((C, C // 2), lambda: (0, 0)),
            pl.BlockSpec((C, 1), lambda: (0, 0)),
            pl.BlockSpec((NUM_CLASSES, C), lambda: (0, 0)),
            pl.BlockSpec((1, NUM_CLASSES), lambda: (0, 0)),
            pl.BlockSpec((1, 2), lambda: (0, 0)),
        ],
        out_specs=pl.BlockSpec((B * P, NUM_CLASSES), lambda: (0, 0)),
        out_shape=jax.ShapeDtypeStruct((B * P, NUM_CLASSES), jnp.float32),
    )(coords, featr, W1.astype(jnp.bfloat16), b1.reshape(C // 2, 1),
      W2.astype(jnp.bfloat16), b2.reshape(C, 1),
      Wvp.astype(jnp.bfloat16), bvp.reshape(1, NUM_CLASSES), scal)

    return logits, prior.reshape(2, B, P, NUM_CLASSES)
